# Initial kernel scaffold; baseline (speedup 1.0000x reference)
import jax
import jax.numpy as jnp
from jax.experimental import pallas as pl


def kernel(x, pos, W, b, batch):
    raise NotImplementedError("write your pallas kernel here")


# speedup vs baseline: 2.2916x; 2.2916x over previous
import jax
import jax.numpy as jnp
from jax.experimental import pallas as pl

NSEG = 16
BLK = 8192


def _pool_kernel(x_ref, pos_ref, w1_ref, w2_ref, b_ref, batch_ref, out_ref):
    i = pl.program_id(0)
    nblk = pl.num_programs(0)

    @pl.when(i == 0)
    def _init():
        out_ref[...] = jnp.full((NSEG, 64), -jnp.inf, dtype=jnp.float32)

    z = jnp.dot(x_ref[...], w1_ref[...], preferred_element_type=jnp.float32)
    z = z + jnp.dot(pos_ref[...], w2_ref[...], preferred_element_type=jnp.float32)

    bb = batch_ref[...]
    lo = batch_ref[0, 0]
    hi = batch_ref[BLK - 1, 0]

    for s in range(NSEG):
        @pl.when(jnp.logical_and(lo <= s, s <= hi))
        def _acc(s=s):
            m = bb == s
            v = jnp.max(jnp.where(m, z, -jnp.inf), axis=0, keepdims=True)
            out_ref[s:s + 1, :] = jnp.maximum(out_ref[s:s + 1, :], v)

    @pl.when(i == nblk - 1)
    def _finish():
        acc = out_ref[...]
        res = jnp.maximum(acc + b_ref[...], 0.0)
        out_ref[...] = jnp.where(acc == -jnp.inf, acc, res)


def kernel(x, pos, W, b, batch):
    n = x.shape[0]
    nblk = n // BLK
    w1 = W[:, :61].T
    w2 = W[:, 61:].T
    b2 = b.reshape(1, 64)
    batch2 = batch.astype(jnp.int32).reshape(n, 1)

    return pl.pallas_call(
        _pool_kernel,
        grid=(nblk,),
        in_specs=[
            pl.BlockSpec((BLK, 61), lambda i: (i, 0)),
            pl.BlockSpec((BLK, 3), lambda i: (i, 0)),
            pl.BlockSpec((61, 64), lambda i: (0, 0)),
            pl.BlockSpec((3, 64), lambda i: (0, 0)),
            pl.BlockSpec((1, 64), lambda i: (0, 0)),
            pl.BlockSpec((BLK, 1), lambda i: (i, 0)),
        ],
        out_specs=pl.BlockSpec((NSEG, 64), lambda i: (0, 0)),
        out_shape=jax.ShapeDtypeStruct((NSEG, 64), jnp.float32),
    )(x, pos, w1, w2, b2, batch2)
